# baseline (speedup 1.0000x reference)
import functools

import jax
import jax.numpy as jnp
from jax import lax
from jax.experimental import pallas as pl
from jax.experimental.pallas import tpu as pltpu
from jax.experimental.pallas import tpu_sc as plsc

D_MODEL = 64
SCALE = 8.0
NUM_CORES = 2
NUM_SUBCORES = 16
NUM_WORKERS = NUM_CORES * NUM_SUBCORES
CHUNK = 512
LANES = 16


@functools.lru_cache(maxsize=None)
def _make_kernel(B: int):
    b_per_w = B // NUM_WORKERS
    n_chunks = b_per_w // CHUNK
    mesh = plsc.VectorSubcoreMesh(core_axis_name="c", subcore_axis_name="s")

    @functools.partial(
        pl.kernel,
        mesh=mesh,
        out_type=jax.ShapeDtypeStruct((B // 2, 2 * D_MODEL), jnp.float32),
        scratch_types=[
            pltpu.VMEM((CHUNK,), jnp.int32),
            pltpu.VMEM((CHUNK,), jnp.int32),
            pltpu.VMEM((CHUNK + LANES,), jnp.int32),
            pltpu.VMEM((CHUNK, 2 * D_MODEL), jnp.float32),
            pltpu.VMEM((CHUNK // 2, 2 * D_MODEL), jnp.float32),
            pltpu.SemaphoreType.DMA,
        ],
    )
    def emb(x_hbm, t2_hbm, out_hbm, idx_v, q_v, p_v, rows_v, o2_v, sem):
        wid = lax.axis_index("s") * NUM_CORES + lax.axis_index("c")
        base = wid * b_per_w
        obase = wid * (b_per_w // 2)

        def chunk_body(c, carry):
            off = pl.multiple_of(base + c * CHUNK, 8)
            pltpu.sync_copy(x_hbm.at[pl.ds(off, CHUNK)], idx_v)
            for k in range(CHUNK // LANES):
                sl = pl.ds(k * LANES, LANES)
                v = idx_v[sl]
                q_v[sl] = lax.shift_right_logical(v, 1)
                p_v[sl] = lax.shift_left(lax.bitwise_and(v, 1), 6)
            pltpu.async_copy(t2_hbm.at[q_v], rows_v, sem).wait()

            def pair_body(m, carry2):
                pv = p_v[pl.ds(2 * m, LANES)]
                o0 = pv[0]
                o1 = pv[1]
                for j in range(D_MODEL // LANES):
                    o2_v[m, pl.ds(j * LANES, LANES)] = (
                        rows_v[2 * m, pl.ds(o0 + j * LANES, LANES)] * SCALE
                    )
                    o2_v[m, pl.ds(D_MODEL + j * LANES, LANES)] = (
                        rows_v[2 * m + 1, pl.ds(o1 + j * LANES, LANES)] * SCALE
                    )
                return carry2

            lax.fori_loop(0, CHUNK // 2, pair_body, 0)
            oo = pl.multiple_of(obase + c * (CHUNK // 2), 8)
            pltpu.sync_copy(o2_v, out_hbm.at[pl.ds(oo, CHUNK // 2)])
            return carry

        lax.fori_loop(0, n_chunks, chunk_body, 0)

    return emb


def kernel(x, table):
    B = x.size
    t2 = table.reshape(table.size // (2 * D_MODEL), 2 * D_MODEL)
    o2 = _make_kernel(B)(x.reshape(-1), t2)
    return o2.reshape(*x.shape, D_MODEL)


# speedup vs baseline: 1.6257x; 1.6257x over previous
import functools

import jax
import jax.numpy as jnp
from jax import lax
from jax.experimental import pallas as pl
from jax.experimental.pallas import tpu as pltpu
from jax.experimental.pallas import tpu_sc as plsc

D_MODEL = 64
SCALE = 8.0
NUM_CORES = 2
NUM_SUBCORES = 16
NUM_WORKERS = NUM_CORES * NUM_SUBCORES
CHUNK = 256
PAIRS_PER_ITER = 2
LANES = 16


@functools.lru_cache(maxsize=None)
def _make_kernel(B: int, V: int):
    b_per_w = B // NUM_WORKERS
    n_chunks = b_per_w // CHUNK
    mesh = plsc.VectorSubcoreMesh(core_axis_name="c", subcore_axis_name="s")

    @functools.partial(
        pl.kernel,
        mesh=mesh,
        out_type=jax.ShapeDtypeStruct((B // 2, 2 * D_MODEL), jnp.float32),
        scratch_types=[
            pltpu.VMEM((b_per_w,), jnp.int32),
            pltpu.VMEM((CHUNK, D_MODEL), jnp.float32),
            pltpu.VMEM((CHUNK, D_MODEL), jnp.float32),
            pltpu.VMEM((CHUNK // 2, 2 * D_MODEL), jnp.float32),
            pltpu.VMEM((CHUNK // 2, 2 * D_MODEL), jnp.float32),
            pltpu.SemaphoreType.DMA,
            pltpu.SemaphoreType.DMA,
        ],
        compiler_params=pltpu.CompilerParams(use_tc_tiling_on_sc=False),
    )
    def emb(x_hbm, t_hbm, out_hbm, idx_all, rows0, rows1, o20, o21, sem0, sem1):
        wid = lax.axis_index("s") * NUM_CORES + lax.axis_index("c")
        base = pl.multiple_of(wid * b_per_w, 8)
        obase = pl.multiple_of(wid * (b_per_w // 2), 8)
        rows = (rows0, rows1)
        o2s = (o20, o21)
        sems = (sem0, sem1)

        pltpu.sync_copy(x_hbm.at[pl.ds(base, b_per_w)], idx_all)
        pltpu.async_copy(t_hbm.at[idx_all.at[pl.ds(0, CHUNK)]], rows0, sem0)

        def super_body(h, carry):
            for b in range(2):
                g = 2 * h + b

                @pl.when(g + 1 < n_chunks)
                def _():
                    nxt = pl.multiple_of((g + 1) * CHUNK, 8)
                    pltpu.async_copy(
                        t_hbm.at[idx_all.at[pl.ds(nxt, CHUNK)]],
                        rows[1 - b],
                        sems[1 - b],
                    )

                goff = pl.multiple_of(g * CHUNK, 8)
                pltpu.make_async_copy(
                    t_hbm.at[idx_all.at[pl.ds(goff, CHUNK)]], rows[b], sems[b]
                ).wait()

                def pack_body(i, carry2):
                    for u in range(PAIRS_PER_ITER):
                        m = i * PAIRS_PER_ITER + u
                        for j in range(D_MODEL // LANES):
                            sl = pl.ds(j * LANES, LANES)
                            o2s[b][m, sl] = rows[b][2 * m, sl] * SCALE
                            sl2 = pl.ds(D_MODEL + j * LANES, LANES)
                            o2s[b][m, sl2] = rows[b][2 * m + 1, sl] * SCALE
                    return carry2

                lax.fori_loop(0, CHUNK // (2 * PAIRS_PER_ITER), pack_body, 0)
                oo = pl.multiple_of(obase + g * (CHUNK // 2), 8)
                pltpu.sync_copy(o2s[b], out_hbm.at[pl.ds(oo, CHUNK // 2)])
            return carry

        lax.fori_loop(0, n_chunks // 2, super_body, 0)

    return emb


def kernel(x, table):
    B = x.size
    V = table.shape[0]
    t_flat = lax.optimization_barrier(table.reshape(-1))
    t_lin = t_flat.reshape(V, D_MODEL)
    o2 = _make_kernel(B, V)(x.reshape(-1), t_lin)
    return o2.reshape(*x.shape, D_MODEL)
